# baseline (speedup 1.0000x reference)
import functools

import jax
import jax.numpy as jnp
from jax import lax
from jax.experimental import pallas as pl
from jax.experimental.pallas import tpu as pltpu
from jax.experimental.pallas import tpu_sc as plsc

N = 10000
E = 160000
D = 256
H = 8
C = 32
NC, NS = 2, 16
K = 80
EPAD = 163840
EPT = EPAD // NS
NCH = EPT // K
NP = N + 8
RPT = 624
RLAST = N - 15 * RPT
BN = 1000
NB = N // BN
F32 = jnp.float32


def _proj_body(xu, xi, wu, wi, vu, vi, hu3, hi3, su, si):
    a = jnp.dot(xu[...], wu[...], preferred_element_type=F32)
    hu3[...] = jnp.stack([a[:, :128], a[:, 128:]])
    su[...] = jnp.dot(xu[...], vu[...], preferred_element_type=F32)
    b = jnp.dot(xi[...], wi[...], preferred_element_type=F32)
    hi3[...] = jnp.stack([b[:, :128], b[:, 128:]])
    si[...] = jnp.dot(xi[...], vi[...], preferred_element_type=F32)


_proj = pl.pallas_call(
    _proj_body,
    grid=(NB,),
    in_specs=[
        pl.BlockSpec((BN, D), lambda i: (i, 0)),
        pl.BlockSpec((BN, D), lambda i: (i, 0)),
        pl.BlockSpec((D, D), lambda i: (0, 0)),
        pl.BlockSpec((D, D), lambda i: (0, 0)),
        pl.BlockSpec((D, 16), lambda i: (0, 0)),
        pl.BlockSpec((D, 16), lambda i: (0, 0)),
    ],
    out_specs=[
        pl.BlockSpec((2, BN, 128), lambda i: (0, i, 0)),
        pl.BlockSpec((2, BN, 128), lambda i: (0, i, 0)),
        pl.BlockSpec((BN, 16), lambda i: (i, 0)),
        pl.BlockSpec((BN, 16), lambda i: (i, 0)),
    ],
    out_shape=[
        jax.ShapeDtypeStruct((2, N, 128), F32),
        jax.ShapeDtypeStruct((2, N, 128), F32),
        jax.ShapeDtypeStruct((N, 16), F32),
        jax.ShapeDtypeStruct((N, 16), F32),
    ],
)


def _ae_body(ea1, ea2, w1, w2, ae1, ae2):
    ae1[...] = jnp.dot(ea1[...], w1[...], preferred_element_type=F32)
    ae2[...] = jnp.dot(ea2[...], w2[...], preferred_element_type=F32)


EG = EPAD // 8
EGB = EG // 8

_ae = pl.pallas_call(
    _ae_body,
    grid=(8,),
    in_specs=[
        pl.BlockSpec((EGB, 128), lambda i: (i, 0)),
        pl.BlockSpec((EGB, 128), lambda i: (i, 0)),
        pl.BlockSpec((128, 128), lambda i: (0, 0)),
        pl.BlockSpec((128, 128), lambda i: (0, 0)),
    ],
    out_specs=[
        pl.BlockSpec((EGB, 128), lambda i: (i, 0)),
        pl.BlockSpec((EGB, 128), lambda i: (i, 0)),
    ],
    out_shape=[
        jax.ShapeDtypeStruct((EG, 128), F32),
        jax.ShapeDtypeStruct((EG, 128), F32),
    ],
)


def _epi_one(acc3, den2, r, bias, g, b):
    out = jnp.concatenate([acc3[0], acc3[1]], axis=-1)
    denb = jnp.dot(den2, r, preferred_element_type=F32) + 1e-16
    out = out / denb + bias
    mu = jnp.mean(out, axis=-1, keepdims=True)
    var = jnp.mean((out - mu) ** 2, axis=-1, keepdims=True)
    y = (out - mu) * lax.rsqrt(var + 1e-5) * g + b
    return jnp.maximum(y, 0.0)


def _epi_body(accu, denu, acci, deni, r, bu, gu, bbu, bi, gi, bbi, xu_o, xi_o):
    xu_o[...] = _epi_one(accu[...], denu[...], r[...], bu[...], gu[...], bbu[...])
    xi_o[...] = _epi_one(acci[...], deni[...], r[...], bi[...], gi[...], bbi[...])


_epi = pl.pallas_call(
    _epi_body,
    grid=(NB,),
    in_specs=[
        pl.BlockSpec((2, BN, 128), lambda i: (0, i, 0)),
        pl.BlockSpec((BN, 16), lambda i: (i, 0)),
        pl.BlockSpec((2, BN, 128), lambda i: (0, i, 0)),
        pl.BlockSpec((BN, 16), lambda i: (i, 0)),
        pl.BlockSpec((16, D), lambda i: (0, 0)),
        pl.BlockSpec((1, D), lambda i: (0, 0)),
        pl.BlockSpec((1, D), lambda i: (0, 0)),
        pl.BlockSpec((1, D), lambda i: (0, 0)),
        pl.BlockSpec((1, D), lambda i: (0, 0)),
        pl.BlockSpec((1, D), lambda i: (0, 0)),
        pl.BlockSpec((1, D), lambda i: (0, 0)),
    ],
    out_specs=[
        pl.BlockSpec((BN, D), lambda i: (i, 0)),
        pl.BlockSpec((BN, D), lambda i: (i, 0)),
    ],
    out_shape=[
        jax.ShapeDtypeStruct((N, D), F32),
        jax.ShapeDtypeStruct((N, D), F32),
    ],
)


def _sc_gat_body(src_h, dst_h, ae_h, ss_h, sd_h, h2n_h, z128_h, z16_h,
                 acc_out, den_out,
                 idx_src0, idx_src1, idx_dst0, idx_dst1, idx_g0, idx_g1,
                 a_s0, a_s1, a_d0, a_d1, a_e0, a_e1, exb0, exb1, buf0, buf1,
                 acc, den,
                 sem_lin0, sem_lin1, sem_gat0, sem_gat1, sem_sc0, sem_sc1):
    idx_src = [idx_src0, idx_src1]
    idx_dst = [idx_dst0, idx_dst1]
    idx_g = [idx_g0, idx_g1]
    a_s = [a_s0, a_s1]
    a_d = [a_d0, a_d1]
    a_e = [a_e0, a_e1]
    exb = [exb0, exb1]
    buf = [buf0, buf1]
    sem_lin = [sem_lin0, sem_lin1]
    sem_gat = [sem_gat0, sem_gat1]
    sem_sc = [sem_sc0, sem_sc1]
    c = lax.axis_index("c")
    s = lax.axis_index("s")
    iota = lax.iota(jnp.int32, 16)

    @pl.when(s < 15)
    def _():
        pltpu.sync_copy(z128_h.at[pl.ds(0, RPT)], acc.at[pl.ds(s * RPT, RPT)])
        pltpu.sync_copy(z16_h.at[pl.ds(0, RPT)], den.at[pl.ds(s * RPT, RPT)])

    @pl.when(s == 15)
    def _():
        pltpu.sync_copy(z128_h.at[pl.ds(0, RLAST + 8)],
                        acc.at[pl.ds(15 * RPT, RLAST + 8)])
        pltpu.sync_copy(z16_h.at[pl.ds(0, RLAST + 8)],
                        den.at[pl.ds(15 * RPT, RLAST + 8)])

    plsc.subcore_barrier()

    hb = 4 * c
    coff = c * N
    base0 = s * EPT
    perm8 = jnp.bitwise_and(iota + 8, 15)
    mask8 = iota < 8
    hvecs = [jnp.full((16,), hb + hh, jnp.int32) for hh in range(4)]

    def _lin_start(i, b):
        base = base0 + i * K
        pltpu.async_copy(src_h.at[pl.ds(base, K)], idx_src[b], sem_lin[b])
        pltpu.async_copy(dst_h.at[pl.ds(base, K)], idx_dst[b], sem_lin[b])
        pltpu.async_copy(ae_h.at[pl.ds(base, K)], a_e[b], sem_lin[b])

    def _lin_wait(b):
        pltpu.make_async_copy(src_h.at[pl.ds(base0, K)], idx_src[b],
                              sem_lin[b]).wait()
        pltpu.make_async_copy(dst_h.at[pl.ds(base0, K)], idx_dst[b],
                              sem_lin[b]).wait()
        pltpu.make_async_copy(ae_h.at[pl.ds(base0, K)], a_e[b],
                              sem_lin[b]).wait()

    def _scat_wait(b):
        pltpu.make_async_copy(exb[b], den.at[idx_dst[b]], sem_sc[b]).wait()
        pltpu.make_async_copy(buf[b], acc.at[idx_dst[b]], sem_sc[b]).wait()

    _lin_start(0, 0)

    def _outer(g, carry):
        for b in range(2):
            i = 2 * g + b
            _lin_wait(b)
            for v in range(K // 16):
                idx_g[b][pl.ds(v * 16, 16)] = (
                    idx_src[b][pl.ds(v * 16, 16)] + coff)
            pltpu.async_copy(ss_h.at[idx_src[b]], a_s[b], sem_gat[b])
            pltpu.async_copy(sd_h.at[idx_dst[b]], a_d[b], sem_gat[b])
            pltpu.async_copy(h2n_h.at[idx_g[b]], buf[b], sem_gat[b])

            @pl.when(i >= 1)
            def _():
                _scat_wait(1 - b)

            @pl.when(i + 1 < NCH)
            def _():
                _lin_start(i + 1, 1 - b)

            pltpu.make_async_copy(ss_h.at[idx_src[b]], a_s[b],
                                  sem_gat[b]).wait()
            pltpu.make_async_copy(sd_h.at[idx_dst[b]], a_d[b],
                                  sem_gat[b]).wait()
            pltpu.make_async_copy(h2n_h.at[idx_g[b]], buf[b],
                                  sem_gat[b]).wait()

            def _edge(j, carry2):
                vd = jnp.take_along_axis(a_d[b][j, :], perm8, axis=0)
                t = a_s[b][j, :] + vd + a_e[b][j, :]
                t = jnp.maximum(t, 0.2 * t)
                e = jnp.where(mask8, jnp.exp(t), 0.0)
                exb[b][j, :] = e
                for hh in range(4):
                    svv = jnp.take_along_axis(e, hvecs[hh], axis=0)
                    for q in range(2):
                        sl = pl.ds(hh * 32 + q * 16, 16)
                        buf[b][j, sl] = buf[b][j, sl] * svv
                return carry2

            lax.fori_loop(0, K, _edge, 0, unroll=4)

            pltpu.async_copy(exb[b], den.at[idx_dst[b]], sem_sc[b], add=True)
            pltpu.async_copy(buf[b], acc.at[idx_dst[b]], sem_sc[b], add=True)
        return carry

    lax.fori_loop(0, NCH // 2, _outer, 0)
    _scat_wait(1)
    plsc.subcore_barrier()

    @pl.when(s < 15)
    def _():
        pltpu.sync_copy(acc.at[pl.ds(s * RPT, RPT)],
                        acc_out.at[pl.ds(coff + s * RPT, RPT)])

        @pl.when(c == 0)
        def _():
            pltpu.sync_copy(den.at[pl.ds(s * RPT, RPT)],
                            den_out.at[pl.ds(s * RPT, RPT)])

    @pl.when(s == 15)
    def _():
        pltpu.sync_copy(acc.at[pl.ds(15 * RPT, RLAST)],
                        acc_out.at[pl.ds(coff + 15 * RPT, RLAST)])

        @pl.when(c == 0)
        def _():
            pltpu.sync_copy(den.at[pl.ds(15 * RPT, RLAST)],
                            den_out.at[pl.ds(15 * RPT, RLAST)])


_sc_gat = functools.partial(
    pl.kernel,
    out_type=[
        jax.ShapeDtypeStruct((2 * N, 128), F32),
        jax.ShapeDtypeStruct((N, 16), F32),
    ],
    mesh=plsc.VectorSubcoreMesh(core_axis_name="c", subcore_axis_name="s",
                                num_cores=NC, num_subcores=NS),
    compiler_params=pltpu.CompilerParams(use_tc_tiling_on_sc=False),
    scratch_types=(
        [pltpu.VMEM((K,), jnp.int32)] * 6
        + [pltpu.VMEM((K, 16), F32)] * 8
        + [pltpu.VMEM((K, 128), F32)] * 2
        + [pltpu.VMEM_SHARED((NP, 128), F32),
           pltpu.VMEM_SHARED((NP, 16), F32)]
        + [pltpu.SemaphoreType.DMA] * 6
    ),
)(_sc_gat_body)


def _fold(w, a):
    return jnp.einsum('dhc,hc->dh', w.reshape(w.shape[0], H, C), a)


def kernel(x_user, x_item, edge_attr_u2i, edge_attr_i2u, params,
           edge_index_u2i, edge_index_i2u):
    i32 = jnp.int32
    pad_e = EPAD - E
    src_u2i = jnp.concatenate([edge_index_u2i[0].astype(i32),
                               jnp.zeros((pad_e,), i32)])
    dst_u2i = jnp.concatenate([edge_index_u2i[1].astype(i32),
                               jnp.full((pad_e,), N, i32)])
    src_i2u = jnp.concatenate([edge_index_i2u[0].astype(i32),
                               jnp.zeros((pad_e,), i32)])
    dst_i2u = jnp.concatenate([edge_index_i2u[1].astype(i32),
                               jnp.full((pad_e,), N, i32)])
    ea_u2i = jnp.pad(edge_attr_u2i, ((0, pad_e), (0, 0)))
    ea_i2u = jnp.pad(edge_attr_i2u, ((0, pad_e), (0, 0)))
    z128 = jnp.zeros((RLAST + 8, 128), F32)
    z16 = jnp.zeros((RLAST + 8, 16), F32)
    r_mat = jnp.concatenate(
        [jnp.kron(jnp.eye(H, dtype=F32), jnp.ones((1, C), F32)),
         jnp.zeros((8, D), F32)], axis=0)

    xu, xi = x_user, x_item
    for lp in params["layers"]:
        pu, pi = lp["u2i"], lp["i2u"]
        v2u = jnp.concatenate([_fold(pu["W_src"], pu["a_src"]),
                               _fold(pi["W_dst"], pi["a_dst"])], axis=1)
        v2i = jnp.concatenate([_fold(pi["W_src"], pi["a_src"]),
                               _fold(pu["W_dst"], pu["a_dst"])], axis=1)
        eye8 = jnp.eye(8, dtype=F32)
        ve_u2i = jnp.kron(eye8, jnp.pad(_fold(pu["W_edge"], pu["a_edge"]),
                                        ((0, 0), (0, 8))))
        ve_i2u = jnp.kron(eye8, jnp.pad(_fold(pi["W_edge"], pi["a_edge"]),
                                        ((0, 0), (0, 8))))

        hu3, hi3, su, si = _proj(xu, xi, pu["W_src"], pi["W_src"], v2u, v2i)
        ae1, ae2 = _ae(ea_u2i.reshape(EG, 128), ea_i2u.reshape(EG, 128),
                       ve_u2i, ve_i2u)
        ae1 = ae1.reshape(EPAD, 16)
        ae2 = ae2.reshape(EPAD, 16)
        hu2n = hu3.reshape(2 * N, 128)
        hi2n = hi3.reshape(2 * N, 128)
        su_p = jnp.pad(su, ((0, 8), (0, 0)))
        si_p = jnp.pad(si, ((0, 8), (0, 0)))

        acc_i, den_i = _sc_gat(src_u2i, dst_u2i, ae1, su, si_p, hu2n,
                               z128, z16)
        acc_u, den_u = _sc_gat(src_i2u, dst_i2u, ae2, si, su_p, hi2n,
                               z128, z16)

        bu = pi["bias"].reshape(1, D)
        bi = pu["bias"].reshape(1, D)
        xu, xi = _epi(acc_u.reshape(2, N, 128), den_u,
                      acc_i.reshape(2, N, 128), den_i,
                      r_mat,
                      bu, lp["g_user"].reshape(1, D), lp["b_user"].reshape(1, D),
                      bi, lp["g_item"].reshape(1, D), lp["b_item"].reshape(1, D))

    return jnp.concatenate([xu, xi], axis=0)


# speedup vs baseline: 1.1334x; 1.1334x over previous
import functools

import jax
import jax.numpy as jnp
from jax import lax
from jax.experimental import pallas as pl
from jax.experimental.pallas import tpu as pltpu
from jax.experimental.pallas import tpu_sc as plsc

N = 10000
E = 160000
D = 256
H = 8
C = 32
NC, NS = 2, 16
K = 80
EPAD = 163840
EPT = EPAD // NS
NCH = EPT // K
NP = N + 8
RPT = 624
RLAST = N - 15 * RPT
BN = 1000
NB = N // BN
F32 = jnp.float32


def _proj_body(xu, xi, wu, wi, vu, vi, hu3, hi3, su, si):
    a = jnp.dot(xu[...], wu[...], preferred_element_type=F32)
    hu3[...] = jnp.stack([a[:, :128], a[:, 128:]])
    su[...] = jnp.dot(xu[...], vu[...], preferred_element_type=F32)
    b = jnp.dot(xi[...], wi[...], preferred_element_type=F32)
    hi3[...] = jnp.stack([b[:, :128], b[:, 128:]])
    si[...] = jnp.dot(xi[...], vi[...], preferred_element_type=F32)


_proj = pl.pallas_call(
    _proj_body,
    grid=(NB,),
    in_specs=[
        pl.BlockSpec((BN, D), lambda i: (i, 0)),
        pl.BlockSpec((BN, D), lambda i: (i, 0)),
        pl.BlockSpec((D, D), lambda i: (0, 0)),
        pl.BlockSpec((D, D), lambda i: (0, 0)),
        pl.BlockSpec((D, 16), lambda i: (0, 0)),
        pl.BlockSpec((D, 16), lambda i: (0, 0)),
    ],
    out_specs=[
        pl.BlockSpec((2, BN, 128), lambda i: (0, i, 0)),
        pl.BlockSpec((2, BN, 128), lambda i: (0, i, 0)),
        pl.BlockSpec((BN, 16), lambda i: (i, 0)),
        pl.BlockSpec((BN, 16), lambda i: (i, 0)),
    ],
    out_shape=[
        jax.ShapeDtypeStruct((2, N, 128), F32),
        jax.ShapeDtypeStruct((2, N, 128), F32),
        jax.ShapeDtypeStruct((N, 16), F32),
        jax.ShapeDtypeStruct((N, 16), F32),
    ],
)


def _ae_body(ea1, ea2, w1, w2, ae1, ae2):
    ae1[...] = jnp.dot(ea1[...], w1[...], preferred_element_type=F32)
    ae2[...] = jnp.dot(ea2[...], w2[...], preferred_element_type=F32)


EG = EPAD // 8
EGB = EG // 8

_ae = pl.pallas_call(
    _ae_body,
    grid=(8,),
    in_specs=[
        pl.BlockSpec((EGB, 128), lambda i: (i, 0)),
        pl.BlockSpec((EGB, 128), lambda i: (i, 0)),
        pl.BlockSpec((128, 128), lambda i: (0, 0)),
        pl.BlockSpec((128, 128), lambda i: (0, 0)),
    ],
    out_specs=[
        pl.BlockSpec((EGB, 128), lambda i: (i, 0)),
        pl.BlockSpec((EGB, 128), lambda i: (i, 0)),
    ],
    out_shape=[
        jax.ShapeDtypeStruct((EG, 128), F32),
        jax.ShapeDtypeStruct((EG, 128), F32),
    ],
)


def _epi_one(acc3, den2, r, bias, g, b):
    out = jnp.concatenate([acc3[0], acc3[1]], axis=-1)
    denb = jnp.dot(den2, r, preferred_element_type=F32) + 1e-16
    out = out / denb + bias
    mu = jnp.mean(out, axis=-1, keepdims=True)
    var = jnp.mean((out - mu) ** 2, axis=-1, keepdims=True)
    y = (out - mu) * lax.rsqrt(var + 1e-5) * g + b
    return jnp.maximum(y, 0.0)


def _epi_body(accu, denu, acci, deni, r, bu, gu, bbu, bi, gi, bbi, xu_o, xi_o):
    xu_o[...] = _epi_one(accu[...], denu[...], r[...], bu[...], gu[...], bbu[...])
    xi_o[...] = _epi_one(acci[...], deni[...], r[...], bi[...], gi[...], bbi[...])


_epi = pl.pallas_call(
    _epi_body,
    grid=(NB,),
    in_specs=[
        pl.BlockSpec((2, BN, 128), lambda i: (0, i, 0)),
        pl.BlockSpec((BN, 16), lambda i: (i, 0)),
        pl.BlockSpec((2, BN, 128), lambda i: (0, i, 0)),
        pl.BlockSpec((BN, 16), lambda i: (i, 0)),
        pl.BlockSpec((16, D), lambda i: (0, 0)),
        pl.BlockSpec((1, D), lambda i: (0, 0)),
        pl.BlockSpec((1, D), lambda i: (0, 0)),
        pl.BlockSpec((1, D), lambda i: (0, 0)),
        pl.BlockSpec((1, D), lambda i: (0, 0)),
        pl.BlockSpec((1, D), lambda i: (0, 0)),
        pl.BlockSpec((1, D), lambda i: (0, 0)),
    ],
    out_specs=[
        pl.BlockSpec((BN, D), lambda i: (i, 0)),
        pl.BlockSpec((BN, D), lambda i: (i, 0)),
    ],
    out_shape=[
        jax.ShapeDtypeStruct((N, D), F32),
        jax.ShapeDtypeStruct((N, D), F32),
    ],
)


def _sc_gat_body(src_h, dst_h, ae_h, ss_h, sd_h, h2n_h, z128_h, z16_h,
                 acc_out, den_out,
                 idx_src0, idx_src1, idx_dst0, idx_dst1, idx_g0, idx_g1,
                 a_s0, a_s1, a_d0, a_d1, a_e0, a_e1, exb0, exb1, buf0, buf1,
                 acc, den,
                 sem_lin0, sem_lin1, sem_gat0, sem_gat1, sem_sc0, sem_sc1):
    idx_src = [idx_src0, idx_src1]
    idx_dst = [idx_dst0, idx_dst1]
    idx_g = [idx_g0, idx_g1]
    a_s = [a_s0, a_s1]
    a_d = [a_d0, a_d1]
    a_e = [a_e0, a_e1]
    exb = [exb0, exb1]
    buf = [buf0, buf1]
    sem_lin = [sem_lin0, sem_lin1]
    sem_gat = [sem_gat0, sem_gat1]
    sem_sc = [sem_sc0, sem_sc1]
    c = lax.axis_index("c")
    s = lax.axis_index("s")
    iota = lax.iota(jnp.int32, 16)

    @pl.when(s < 15)
    def _():
        pltpu.sync_copy(z128_h.at[pl.ds(0, RPT)], acc.at[pl.ds(s * RPT, RPT)])
        pltpu.sync_copy(z16_h.at[pl.ds(0, RPT)], den.at[pl.ds(s * RPT, RPT)])

    @pl.when(s == 15)
    def _():
        pltpu.sync_copy(z128_h.at[pl.ds(0, RLAST + 8)],
                        acc.at[pl.ds(15 * RPT, RLAST + 8)])
        pltpu.sync_copy(z16_h.at[pl.ds(0, RLAST + 8)],
                        den.at[pl.ds(15 * RPT, RLAST + 8)])

    plsc.subcore_barrier()

    hb = 4 * c
    coff = c * N
    base0 = s * EPT
    perm8 = jnp.bitwise_and(iota + 8, 15)
    mask8 = iota < 8
    hvecs = [jnp.full((16,), hb + hh, jnp.int32) for hh in range(4)]

    def _lin_start(i, b):
        base = base0 + i * K
        pltpu.async_copy(src_h.at[pl.ds(base, K)], idx_src[b], sem_lin[b])
        pltpu.async_copy(dst_h.at[pl.ds(base, K)], idx_dst[b], sem_lin[b])
        pltpu.async_copy(ae_h.at[pl.ds(base, K)], a_e[b], sem_lin[b])

    def _lin_wait(b):
        pltpu.make_async_copy(src_h.at[pl.ds(base0, K)], idx_src[b],
                              sem_lin[b]).wait()
        pltpu.make_async_copy(dst_h.at[pl.ds(base0, K)], idx_dst[b],
                              sem_lin[b]).wait()
        pltpu.make_async_copy(ae_h.at[pl.ds(base0, K)], a_e[b],
                              sem_lin[b]).wait()

    def _scat_wait(b):
        pltpu.make_async_copy(exb[b], den.at[idx_dst[b]], sem_sc[b]).wait()
        pltpu.make_async_copy(buf[b], acc.at[idx_dst[b]], sem_sc[b]).wait()

    _lin_start(0, 0)

    def _outer(g, carry):
        for b in range(2):
            i = 2 * g + b
            _lin_wait(b)
            for v in range(K // 16):
                idx_g[b][pl.ds(v * 16, 16)] = (
                    idx_src[b][pl.ds(v * 16, 16)] + coff)
            pltpu.async_copy(ss_h.at[idx_src[b]], a_s[b], sem_gat[b])
            pltpu.async_copy(sd_h.at[idx_dst[b]], a_d[b], sem_gat[b])
            pltpu.async_copy(h2n_h.at[idx_g[b]], buf[b], sem_gat[b])

            @pl.when(i >= 1)
            def _():
                _scat_wait(1 - b)

            @pl.when(i + 1 < NCH)
            def _():
                _lin_start(i + 1, 1 - b)

            pltpu.make_async_copy(ss_h.at[idx_src[b]], a_s[b],
                                  sem_gat[b]).wait()
            pltpu.make_async_copy(sd_h.at[idx_dst[b]], a_d[b],
                                  sem_gat[b]).wait()

            def _exrow(j, carry2):
                vd = jnp.take_along_axis(a_d[b][j, :], perm8, axis=0)
                t = a_s[b][j, :] + vd + a_e[b][j, :]
                t = jnp.maximum(t, 0.2 * t)
                exb[b][j, :] = jnp.where(mask8, jnp.exp(t), 0.0)
                return carry2

            lax.fori_loop(0, K, _exrow, 0, unroll=4)

            pltpu.make_async_copy(h2n_h.at[idx_g[b]], buf[b],
                                  sem_gat[b]).wait()

            def _scale(j, carry2):
                exr = exb[b][j, :]
                for hh in range(4):
                    svv = jnp.take_along_axis(exr, hvecs[hh], axis=0)
                    for q in range(2):
                        sl = pl.ds(hh * 32 + q * 16, 16)
                        buf[b][j, sl] = buf[b][j, sl] * svv
                return carry2

            lax.fori_loop(0, K, _scale, 0, unroll=4)

            pltpu.async_copy(exb[b], den.at[idx_dst[b]], sem_sc[b], add=True)
            pltpu.async_copy(buf[b], acc.at[idx_dst[b]], sem_sc[b], add=True)
        return carry

    lax.fori_loop(0, NCH // 2, _outer, 0)
    _scat_wait(1)
    plsc.subcore_barrier()

    @pl.when(s < 15)
    def _():
        pltpu.sync_copy(acc.at[pl.ds(s * RPT, RPT)],
                        acc_out.at[pl.ds(coff + s * RPT, RPT)])

        @pl.when(c == 0)
        def _():
            pltpu.sync_copy(den.at[pl.ds(s * RPT, RPT)],
                            den_out.at[pl.ds(s * RPT, RPT)])

    @pl.when(s == 15)
    def _():
        pltpu.sync_copy(acc.at[pl.ds(15 * RPT, RLAST)],
                        acc_out.at[pl.ds(coff + 15 * RPT, RLAST)])

        @pl.when(c == 0)
        def _():
            pltpu.sync_copy(den.at[pl.ds(15 * RPT, RLAST)],
                            den_out.at[pl.ds(15 * RPT, RLAST)])


_sc_gat = functools.partial(
    pl.kernel,
    out_type=[
        jax.ShapeDtypeStruct((2 * N, 128), F32),
        jax.ShapeDtypeStruct((N, 16), F32),
    ],
    mesh=plsc.VectorSubcoreMesh(core_axis_name="c", subcore_axis_name="s",
                                num_cores=NC, num_subcores=NS),
    compiler_params=pltpu.CompilerParams(use_tc_tiling_on_sc=False),
    scratch_types=(
        [pltpu.VMEM((K,), jnp.int32)] * 6
        + [pltpu.VMEM((K, 16), F32)] * 8
        + [pltpu.VMEM((K, 128), F32)] * 2
        + [pltpu.VMEM_SHARED((NP, 128), F32),
           pltpu.VMEM_SHARED((NP, 16), F32)]
        + [pltpu.SemaphoreType.DMA] * 6
    ),
)(_sc_gat_body)


def _fold(w, a):
    return jnp.einsum('dhc,hc->dh', w.reshape(w.shape[0], H, C), a)


def kernel(x_user, x_item, edge_attr_u2i, edge_attr_i2u, params,
           edge_index_u2i, edge_index_i2u):
    i32 = jnp.int32
    pad_e = EPAD - E
    src_u2i = jnp.concatenate([edge_index_u2i[0].astype(i32),
                               jnp.zeros((pad_e,), i32)])
    dst_u2i = jnp.concatenate([edge_index_u2i[1].astype(i32),
                               jnp.full((pad_e,), N, i32)])
    src_i2u = jnp.concatenate([edge_index_i2u[0].astype(i32),
                               jnp.zeros((pad_e,), i32)])
    dst_i2u = jnp.concatenate([edge_index_i2u[1].astype(i32),
                               jnp.full((pad_e,), N, i32)])
    ea_u2i = jnp.pad(edge_attr_u2i, ((0, pad_e), (0, 0)))
    ea_i2u = jnp.pad(edge_attr_i2u, ((0, pad_e), (0, 0)))
    z128 = jnp.zeros((RLAST + 8, 128), F32)
    z16 = jnp.zeros((RLAST + 8, 16), F32)
    r_mat = jnp.concatenate(
        [jnp.kron(jnp.eye(H, dtype=F32), jnp.ones((1, C), F32)),
         jnp.zeros((8, D), F32)], axis=0)

    xu, xi = x_user, x_item
    for lp in params["layers"]:
        pu, pi = lp["u2i"], lp["i2u"]
        v2u = jnp.concatenate([_fold(pu["W_src"], pu["a_src"]),
                               _fold(pi["W_dst"], pi["a_dst"])], axis=1)
        v2i = jnp.concatenate([_fold(pi["W_src"], pi["a_src"]),
                               _fold(pu["W_dst"], pu["a_dst"])], axis=1)
        eye8 = jnp.eye(8, dtype=F32)
        ve_u2i = jnp.kron(eye8, jnp.pad(_fold(pu["W_edge"], pu["a_edge"]),
                                        ((0, 0), (0, 8))))
        ve_i2u = jnp.kron(eye8, jnp.pad(_fold(pi["W_edge"], pi["a_edge"]),
                                        ((0, 0), (0, 8))))

        hu3, hi3, su, si = _proj(xu, xi, pu["W_src"], pi["W_src"], v2u, v2i)
        ae1, ae2 = _ae(ea_u2i.reshape(EG, 128), ea_i2u.reshape(EG, 128),
                       ve_u2i, ve_i2u)
        ae1 = ae1.reshape(EPAD, 16)
        ae2 = ae2.reshape(EPAD, 16)
        hu2n = hu3.reshape(2 * N, 128)
        hi2n = hi3.reshape(2 * N, 128)
        su_p = jnp.pad(su, ((0, 8), (0, 0)))
        si_p = jnp.pad(si, ((0, 8), (0, 0)))

        acc_i, den_i = _sc_gat(src_u2i, dst_u2i, ae1, su, si_p, hu2n,
                               z128, z16)
        acc_u, den_u = _sc_gat(src_i2u, dst_i2u, ae2, si, su_p, hi2n,
                               z128, z16)

        bu = pi["bias"].reshape(1, D)
        bi = pu["bias"].reshape(1, D)
        xu, xi = _epi(acc_u.reshape(2, N, 128), den_u,
                      acc_i.reshape(2, N, 128), den_i,
                      r_mat,
                      bu, lp["g_user"].reshape(1, D), lp["b_user"].reshape(1, D),
                      bi, lp["g_item"].reshape(1, D), lp["b_item"].reshape(1, D))

    return jnp.concatenate([xu, xi], axis=0)


# speedup vs baseline: 1.2749x; 1.1248x over previous
import functools

import jax
import jax.numpy as jnp
from jax import lax
from jax.experimental import pallas as pl
from jax.experimental.pallas import tpu as pltpu
from jax.experimental.pallas import tpu_sc as plsc

N = 10000
E = 160000
D = 256
H = 8
C = 32
NC, NS = 2, 16
K = 96
EPAD = 162816
EPT = EPAD // NS
NCH = EPT // K
NP = N + 8
RPT = 624
RLAST = N - 15 * RPT
BN = 1000
NB = N // BN
F32 = jnp.float32


def _proj_body(xu, xi, wu, wi, vu, vi, hu3, hi3, su, si):
    a = jnp.dot(xu[...], wu[...], preferred_element_type=F32)
    hu3[...] = jnp.stack([a[:, :128], a[:, 128:]])
    su[...] = jnp.dot(xu[...], vu[...], preferred_element_type=F32)
    b = jnp.dot(xi[...], wi[...], preferred_element_type=F32)
    hi3[...] = jnp.stack([b[:, :128], b[:, 128:]])
    si[...] = jnp.dot(xi[...], vi[...], preferred_element_type=F32)


_proj = pl.pallas_call(
    _proj_body,
    grid=(NB,),
    in_specs=[
        pl.BlockSpec((BN, D), lambda i: (i, 0)),
        pl.BlockSpec((BN, D), lambda i: (i, 0)),
        pl.BlockSpec((D, D), lambda i: (0, 0)),
        pl.BlockSpec((D, D), lambda i: (0, 0)),
        pl.BlockSpec((D, 16), lambda i: (0, 0)),
        pl.BlockSpec((D, 16), lambda i: (0, 0)),
    ],
    out_specs=[
        pl.BlockSpec((2, BN, 128), lambda i: (0, i, 0)),
        pl.BlockSpec((2, BN, 128), lambda i: (0, i, 0)),
        pl.BlockSpec((BN, 16), lambda i: (i, 0)),
        pl.BlockSpec((BN, 16), lambda i: (i, 0)),
    ],
    out_shape=[
        jax.ShapeDtypeStruct((2, N, 128), F32),
        jax.ShapeDtypeStruct((2, N, 128), F32),
        jax.ShapeDtypeStruct((N, 16), F32),
        jax.ShapeDtypeStruct((N, 16), F32),
    ],
)


def _ae_body(ea1, ea2, w1, w2, ae1, ae2):
    ae1[...] = jnp.dot(ea1[...], w1[...], preferred_element_type=F32)
    ae2[...] = jnp.dot(ea2[...], w2[...], preferred_element_type=F32)


EG = EPAD // 8
EGB = EG // 8

_ae = pl.pallas_call(
    _ae_body,
    grid=(8,),
    in_specs=[
        pl.BlockSpec((EGB, 128), lambda i: (i, 0)),
        pl.BlockSpec((EGB, 128), lambda i: (i, 0)),
        pl.BlockSpec((128, 128), lambda i: (0, 0)),
        pl.BlockSpec((128, 128), lambda i: (0, 0)),
    ],
    out_specs=[
        pl.BlockSpec((EGB, 128), lambda i: (i, 0)),
        pl.BlockSpec((EGB, 128), lambda i: (i, 0)),
    ],
    out_shape=[
        jax.ShapeDtypeStruct((EG, 128), F32),
        jax.ShapeDtypeStruct((EG, 128), F32),
    ],
)


def _epi_one(acc3, den2, r, bias, g, b):
    out = jnp.concatenate([acc3[0], acc3[1]], axis=-1)
    denb = jnp.dot(den2, r, preferred_element_type=F32) + 1e-16
    out = out / denb + bias
    mu = jnp.mean(out, axis=-1, keepdims=True)
    var = jnp.mean((out - mu) ** 2, axis=-1, keepdims=True)
    y = (out - mu) * lax.rsqrt(var + 1e-5) * g + b
    return jnp.maximum(y, 0.0)


def _epi_body(accu, denu, acci, deni, r, bu, gu, bbu, bi, gi, bbi, xu_o, xi_o):
    xu_o[...] = _epi_one(accu[...], denu[...], r[...], bu[...], gu[...], bbu[...])
    xi_o[...] = _epi_one(acci[...], deni[...], r[...], bi[...], gi[...], bbi[...])


_epi = pl.pallas_call(
    _epi_body,
    grid=(NB,),
    in_specs=[
        pl.BlockSpec((2, BN, 128), lambda i: (0, i, 0)),
        pl.BlockSpec((BN, 16), lambda i: (i, 0)),
        pl.BlockSpec((2, BN, 128), lambda i: (0, i, 0)),
        pl.BlockSpec((BN, 16), lambda i: (i, 0)),
        pl.BlockSpec((16, D), lambda i: (0, 0)),
        pl.BlockSpec((1, D), lambda i: (0, 0)),
        pl.BlockSpec((1, D), lambda i: (0, 0)),
        pl.BlockSpec((1, D), lambda i: (0, 0)),
        pl.BlockSpec((1, D), lambda i: (0, 0)),
        pl.BlockSpec((1, D), lambda i: (0, 0)),
        pl.BlockSpec((1, D), lambda i: (0, 0)),
    ],
    out_specs=[
        pl.BlockSpec((BN, D), lambda i: (i, 0)),
        pl.BlockSpec((BN, D), lambda i: (i, 0)),
    ],
    out_shape=[
        jax.ShapeDtypeStruct((N, D), F32),
        jax.ShapeDtypeStruct((N, D), F32),
    ],
)


def _sc_gat_body(src_h, dst_h, ae_h, ss_h, sd_h, h2n_h, z128_h, z16_h,
                 acc_out, den_out,
                 idx_src0, idx_src1, idx_dst0, idx_dst1, idx_g0, idx_g1,
                 a_s0, a_s1, a_d0, a_d1, a_e0, a_e1, buf0, buf1,
                 acc, den,
                 sem_lin0, sem_lin1, sem_gat0, sem_gat1, sem_sc0, sem_sc1):
    idx_src = [idx_src0, idx_src1]
    idx_dst = [idx_dst0, idx_dst1]
    idx_g = [idx_g0, idx_g1]
    a_s = [a_s0, a_s1]
    a_d = [a_d0, a_d1]
    a_e = [a_e0, a_e1]
    exb = a_s
    buf = [buf0, buf1]
    sem_lin = [sem_lin0, sem_lin1]
    sem_gat = [sem_gat0, sem_gat1]
    sem_sc = [sem_sc0, sem_sc1]
    c = lax.axis_index("c")
    s = lax.axis_index("s")
    iota = lax.iota(jnp.int32, 16)

    @pl.when(s < 15)
    def _():
        pltpu.sync_copy(z128_h.at[pl.ds(0, RPT)], acc.at[pl.ds(s * RPT, RPT)])
        pltpu.sync_copy(z16_h.at[pl.ds(0, RPT)], den.at[pl.ds(s * RPT, RPT)])

    @pl.when(s == 15)
    def _():
        pltpu.sync_copy(z128_h.at[pl.ds(0, RLAST + 8)],
                        acc.at[pl.ds(15 * RPT, RLAST + 8)])
        pltpu.sync_copy(z16_h.at[pl.ds(0, RLAST + 8)],
                        den.at[pl.ds(15 * RPT, RLAST + 8)])

    plsc.subcore_barrier()

    hb = 4 * c
    coff = c * N
    base0 = s * EPT
    perm8 = jnp.bitwise_and(iota + 8, 15)
    mask8 = iota < 8
    hvecs = [jnp.full((16,), hb + hh, jnp.int32) for hh in range(4)]

    def _lin_start(i, b):
        base = base0 + i * K
        pltpu.async_copy(src_h.at[pl.ds(base, K)], idx_src[b], sem_lin[b])
        pltpu.async_copy(dst_h.at[pl.ds(base, K)], idx_dst[b], sem_lin[b])
        pltpu.async_copy(ae_h.at[pl.ds(base, K)], a_e[b], sem_lin[b])

    def _lin_wait(b):
        pltpu.make_async_copy(src_h.at[pl.ds(base0, K)], idx_src[b],
                              sem_lin[b]).wait()
        pltpu.make_async_copy(dst_h.at[pl.ds(base0, K)], idx_dst[b],
                              sem_lin[b]).wait()
        pltpu.make_async_copy(ae_h.at[pl.ds(base0, K)], a_e[b],
                              sem_lin[b]).wait()

    def _scat_wait(b):
        pltpu.make_async_copy(exb[b], den.at[idx_dst[b]], sem_sc[b]).wait()
        pltpu.make_async_copy(buf[b], acc.at[idx_dst[b]], sem_sc[b]).wait()

    _lin_start(0, 0)

    def _outer(g, carry):
        for b in range(2):
            i = 2 * g + b
            _lin_wait(b)
            for v in range(K // 16):
                idx_g[b][pl.ds(v * 16, 16)] = (
                    idx_src[b][pl.ds(v * 16, 16)] + coff)
            pltpu.async_copy(ss_h.at[idx_src[b]], a_s[b], sem_gat[b])
            pltpu.async_copy(sd_h.at[idx_dst[b]], a_d[b], sem_gat[b])
            pltpu.async_copy(h2n_h.at[idx_g[b]], buf[b], sem_gat[b])

            @pl.when(i >= 1)
            def _():
                _scat_wait(1 - b)

            @pl.when(i + 1 < NCH)
            def _():
                _lin_start(i + 1, 1 - b)

            pltpu.make_async_copy(ss_h.at[idx_src[b]], a_s[b],
                                  sem_gat[b]).wait()
            pltpu.make_async_copy(sd_h.at[idx_dst[b]], a_d[b],
                                  sem_gat[b]).wait()

            def _exrow(j, carry2):
                vd = jnp.take_along_axis(a_d[b][j, :], perm8, axis=0)
                t = a_s[b][j, :] + vd + a_e[b][j, :]
                t = jnp.maximum(t, 0.2 * t)
                exb[b][j, :] = jnp.where(mask8, jnp.exp(t), 0.0)
                return carry2

            lax.fori_loop(0, K, _exrow, 0, unroll=4)

            pltpu.make_async_copy(h2n_h.at[idx_g[b]], buf[b],
                                  sem_gat[b]).wait()

            def _scale(j, carry2):
                exr = exb[b][j, :]
                for hh in range(4):
                    svv = jnp.take_along_axis(exr, hvecs[hh], axis=0)
                    for q in range(2):
                        sl = pl.ds(hh * 32 + q * 16, 16)
                        buf[b][j, sl] = buf[b][j, sl] * svv
                return carry2

            lax.fori_loop(0, K, _scale, 0, unroll=4)

            pltpu.async_copy(exb[b], den.at[idx_dst[b]], sem_sc[b], add=True)
            pltpu.async_copy(buf[b], acc.at[idx_dst[b]], sem_sc[b], add=True)
        return carry

    lax.fori_loop(0, NCH // 2, _outer, 0)
    _scat_wait(1)
    plsc.subcore_barrier()

    @pl.when(s < 15)
    def _():
        pltpu.sync_copy(acc.at[pl.ds(s * RPT, RPT)],
                        acc_out.at[pl.ds(coff + s * RPT, RPT)])

        @pl.when(c == 0)
        def _():
            pltpu.sync_copy(den.at[pl.ds(s * RPT, RPT)],
                            den_out.at[pl.ds(s * RPT, RPT)])

    @pl.when(s == 15)
    def _():
        pltpu.sync_copy(acc.at[pl.ds(15 * RPT, RLAST)],
                        acc_out.at[pl.ds(coff + 15 * RPT, RLAST)])

        @pl.when(c == 0)
        def _():
            pltpu.sync_copy(den.at[pl.ds(15 * RPT, RLAST)],
                            den_out.at[pl.ds(15 * RPT, RLAST)])


_sc_gat = functools.partial(
    pl.kernel,
    out_type=[
        jax.ShapeDtypeStruct((2 * N, 128), F32),
        jax.ShapeDtypeStruct((N, 16), F32),
    ],
    mesh=plsc.VectorSubcoreMesh(core_axis_name="c", subcore_axis_name="s",
                                num_cores=NC, num_subcores=NS),
    compiler_params=pltpu.CompilerParams(use_tc_tiling_on_sc=False),
    scratch_types=(
        [pltpu.VMEM((K,), jnp.int32)] * 6
        + [pltpu.VMEM((K, 16), F32)] * 6
        + [pltpu.VMEM((K, 128), F32)] * 2
        + [pltpu.VMEM_SHARED((NP, 128), F32),
           pltpu.VMEM_SHARED((NP, 16), F32)]
        + [pltpu.SemaphoreType.DMA] * 6
    ),
)(_sc_gat_body)


def _fold(w, a):
    return jnp.einsum('dhc,hc->dh', w.reshape(w.shape[0], H, C), a)


def kernel(x_user, x_item, edge_attr_u2i, edge_attr_i2u, params,
           edge_index_u2i, edge_index_i2u):
    i32 = jnp.int32
    pad_e = EPAD - E
    src_u2i = jnp.concatenate([edge_index_u2i[0].astype(i32),
                               jnp.zeros((pad_e,), i32)])
    dst_u2i = jnp.concatenate([edge_index_u2i[1].astype(i32),
                               jnp.full((pad_e,), N, i32)])
    src_i2u = jnp.concatenate([edge_index_i2u[0].astype(i32),
                               jnp.zeros((pad_e,), i32)])
    dst_i2u = jnp.concatenate([edge_index_i2u[1].astype(i32),
                               jnp.full((pad_e,), N, i32)])
    ea_u2i = jnp.pad(edge_attr_u2i, ((0, pad_e), (0, 0)))
    ea_i2u = jnp.pad(edge_attr_i2u, ((0, pad_e), (0, 0)))
    z128 = jnp.zeros((RLAST + 8, 128), F32)
    z16 = jnp.zeros((RLAST + 8, 16), F32)
    r_mat = jnp.concatenate(
        [jnp.kron(jnp.eye(H, dtype=F32), jnp.ones((1, C), F32)),
         jnp.zeros((8, D), F32)], axis=0)

    xu, xi = x_user, x_item
    for lp in params["layers"]:
        pu, pi = lp["u2i"], lp["i2u"]
        v2u = jnp.concatenate([_fold(pu["W_src"], pu["a_src"]),
                               _fold(pi["W_dst"], pi["a_dst"])], axis=1)
        v2i = jnp.concatenate([_fold(pi["W_src"], pi["a_src"]),
                               _fold(pu["W_dst"], pu["a_dst"])], axis=1)
        eye8 = jnp.eye(8, dtype=F32)
        ve_u2i = jnp.kron(eye8, jnp.pad(_fold(pu["W_edge"], pu["a_edge"]),
                                        ((0, 0), (0, 8))))
        ve_i2u = jnp.kron(eye8, jnp.pad(_fold(pi["W_edge"], pi["a_edge"]),
                                        ((0, 0), (0, 8))))

        hu3, hi3, su, si = _proj(xu, xi, pu["W_src"], pi["W_src"], v2u, v2i)
        ae1, ae2 = _ae(ea_u2i.reshape(EG, 128), ea_i2u.reshape(EG, 128),
                       ve_u2i, ve_i2u)
        ae1 = ae1.reshape(EPAD, 16)
        ae2 = ae2.reshape(EPAD, 16)
        hu2n = hu3.reshape(2 * N, 128)
        hi2n = hi3.reshape(2 * N, 128)
        su_p = jnp.pad(su, ((0, 8), (0, 0)))
        si_p = jnp.pad(si, ((0, 8), (0, 0)))

        acc_i, den_i = _sc_gat(src_u2i, dst_u2i, ae1, su, si_p, hu2n,
                               z128, z16)
        acc_u, den_u = _sc_gat(src_i2u, dst_i2u, ae2, si, su_p, hi2n,
                               z128, z16)

        bu = pi["bias"].reshape(1, D)
        bi = pu["bias"].reshape(1, D)
        xu, xi = _epi(acc_u.reshape(2, N, 128), den_u,
                      acc_i.reshape(2, N, 128), den_i,
                      r_mat,
                      bu, lp["g_user"].reshape(1, D), lp["b_user"].reshape(1, D),
                      bi, lp["g_item"].reshape(1, D), lp["b_item"].reshape(1, D))

    return jnp.concatenate([xu, xi], axis=0)
